# Initial kernel scaffold; baseline (speedup 1.0000x reference)
import jax
import jax.numpy as jnp
from jax.experimental import pallas as pl


def kernel(x, input_embedding_table):
    raise NotImplementedError("write your pallas kernel here")


# speedup vs baseline: 2.9656x; 2.9656x over previous
import functools

import jax
import jax.numpy as jnp
from jax import lax
from jax.experimental import pallas as pl
from jax.experimental.pallas import tpu as pltpu
from jax.experimental.pallas import tpu_sc as plsc

_D = 128
_NW = 32
_CHUNK = 128


def _make_gather(b_total):
    rows_per_w = b_total // _NW
    n_chunks = rows_per_w // _CHUNK
    mesh = plsc.VectorSubcoreMesh(core_axis_name="c", subcore_axis_name="s")

    @functools.partial(
        pl.kernel,
        mesh=mesh,
        out_type=jax.ShapeDtypeStruct((b_total, _D), jnp.float32),
        scratch_types=[
            pltpu.VMEM((n_chunks, _CHUNK), jnp.int32),
            pltpu.VMEM((_CHUNK, _D), jnp.float32),
            pltpu.SemaphoreType.DMA,
        ],
    )
    def gather(idx_hbm, tbl_hbm, out_hbm, idx_v, rows_v, sem):
        wid = lax.axis_index("s") * 2 + lax.axis_index("c")
        pltpu.sync_copy(idx_hbm.at[wid], idx_v)
        base = wid * rows_per_w

        def body(j, carry):
            pltpu.async_copy(tbl_hbm.at[idx_v.at[j]], rows_v, sem).wait()
            pltpu.sync_copy(rows_v, out_hbm.at[pl.ds(base + j * _CHUNK, _CHUNK)])
            return carry

        lax.fori_loop(0, n_chunks, body, 0)

    return gather


_gather = _make_gather(4096 * 50)


def kernel(x, input_embedding_table):
    b, h = x.shape
    idx = x.reshape(_NW, (b * h) // (_NW * _CHUNK), _CHUNK).astype(jnp.int32)
    out = _gather(idx, input_embedding_table)
    return out.reshape(b, h, _D)


# speedup vs baseline: 3.3453x; 1.1280x over previous
import functools

import jax
import jax.numpy as jnp
from jax import lax
from jax.experimental import pallas as pl
from jax.experimental.pallas import tpu as pltpu
from jax.experimental.pallas import tpu_sc as plsc

_D = 128
_NW = 32
_CHUNK = 128
_NBUF = 5
_LEAD = 2


def _make_gather(b_total):
    rows_per_w = b_total // _NW
    n_chunks = rows_per_w // _CHUNK
    n_groups = n_chunks // _NBUF
    mesh = plsc.VectorSubcoreMesh(core_axis_name="c", subcore_axis_name="s")

    @functools.partial(
        pl.kernel,
        mesh=mesh,
        out_type=jax.ShapeDtypeStruct((b_total, _D), jnp.float32),
        scratch_types=[
            pltpu.VMEM((n_chunks, _CHUNK), jnp.int32),
            pltpu.VMEM((_NBUF, _CHUNK, _D), jnp.float32),
        ]
        + [pltpu.SemaphoreType.DMA] * (2 * _NBUF),
    )
    def gather(idx_hbm, tbl_hbm, out_hbm, idx_v, bufs, *sems):
        gsem = sems[:_NBUF]
        osem = sems[_NBUF:]
        wid = lax.axis_index("s") * 2 + lax.axis_index("c")
        pltpu.sync_copy(idx_hbm.at[wid], idx_v)
        base = wid * rows_per_w

        def fire_gather(j, b):
            pltpu.async_copy(tbl_hbm.at[idx_v.at[j]], bufs.at[b], gsem[b])

        def drain_gather(j, b):
            pltpu.make_async_copy(tbl_hbm.at[idx_v.at[j]], bufs.at[b],
                                  gsem[b]).wait()

        def fire_out(j, b):
            pltpu.async_copy(bufs.at[b],
                             out_hbm.at[pl.ds(base + j * _CHUNK, _CHUNK)],
                             osem[b])

        def drain_out(j, b):
            pltpu.make_async_copy(bufs.at[b],
                                  out_hbm.at[pl.ds(base + j * _CHUNK, _CHUNK)],
                                  osem[b]).wait()

        for b in range(_LEAD):
            fire_gather(b, b)

        def step(j, b, first, last):
            c = (b + _LEAD) % _NBUF
            if not last:
                if not first:
                    drain_out(j + _LEAD - _NBUF, c)
                fire_gather(j + _LEAD, c)
            drain_gather(j, b)
            fire_out(j, b)

        for b in range(_NBUF):
            step(b, b, first=(b + _LEAD < _NBUF), last=False)

        def group(i, carry):
            j0 = i * _NBUF
            for b in range(_NBUF):
                step(j0 + b, b, first=False, last=False)
            return carry

        lax.fori_loop(1, n_groups - 1, group, 0)

        jl = (n_groups - 1) * _NBUF
        for b in range(_NBUF):
            step(jl + b, b, first=False, last=(b + _LEAD >= _NBUF))

        for b in range(_NBUF):
            drain_out(jl + b, b)

    return gather


_gather = _make_gather(4096 * 50)


def kernel(x, input_embedding_table):
    b, h = x.shape
    idx = x.reshape(_NW, (b * h) // (_NW * _CHUNK), _CHUNK).astype(jnp.int32)
    out = _gather(idx, input_embedding_table)
    return out.reshape(b, h, _D)


# speedup vs baseline: 5.9425x; 1.7764x over previous
import functools

import jax
import jax.numpy as jnp
from jax import lax
from jax.experimental import pallas as pl
from jax.experimental.pallas import tpu as pltpu
from jax.experimental.pallas import tpu_sc as plsc

_D = 128
_NW = 32
_NBUF = 4
_LEAD = 2
_BPC = 2


def _make_gather(batch, hist):
    rows_per_w = batch * hist // _NW
    b_per_w = batch // _NW
    chunk = _BPC * hist
    n_chunks = rows_per_w // chunk
    n_groups = n_chunks // _NBUF
    mesh = plsc.VectorSubcoreMesh(core_axis_name="c", subcore_axis_name="s")

    @functools.partial(
        pl.kernel,
        mesh=mesh,
        out_type=jax.ShapeDtypeStruct((batch, hist, _D), jnp.float32),
        scratch_types=[
            pltpu.VMEM((n_chunks, chunk), jnp.int32),
            pltpu.VMEM((_NBUF, chunk, _D), jnp.float32),
        ]
        + [pltpu.SemaphoreType.DMA] * (2 * _NBUF),
    )
    def gather(idx_hbm, tbl_hbm, out_hbm, idx_v, bufs, *sems):
        gsem = sems[:_NBUF]
        osem = sems[_NBUF:]
        wid = lax.axis_index("s") * 2 + lax.axis_index("c")
        pltpu.sync_copy(idx_hbm.at[wid], idx_v)
        b_base = wid * b_per_w

        def fire_gather(j, b):
            pltpu.async_copy(tbl_hbm.at[idx_v.at[j]], bufs.at[b], gsem[b])

        def drain_gather(j, b):
            pltpu.make_async_copy(tbl_hbm.at[idx_v.at[j]], bufs.at[b],
                                  gsem[b]).wait()

        def fire_out(j, b):
            for r in range(_BPC):
                pltpu.async_copy(bufs.at[b, pl.ds(r * hist, hist)],
                                 out_hbm.at[b_base + j * _BPC + r],
                                 osem[b])

        def drain_out(j, b):
            for r in range(_BPC):
                pltpu.make_async_copy(bufs.at[b, pl.ds(r * hist, hist)],
                                      out_hbm.at[b_base + j * _BPC + r],
                                      osem[b]).wait()

        for b in range(_LEAD):
            fire_gather(b, b)

        def step(j, b, first, last):
            c = (b + _LEAD) % _NBUF
            if not last:
                if not first:
                    drain_out(j + _LEAD - _NBUF, c)
                fire_gather(j + _LEAD, c)
            drain_gather(j, b)
            fire_out(j, b)

        for b in range(_NBUF):
            step(b, b, first=(b + _LEAD < _NBUF), last=False)

        def group(i, carry):
            j0 = i * _NBUF
            for b in range(_NBUF):
                step(j0 + b, b, first=False, last=False)
            return carry

        lax.fori_loop(1, n_groups - 1, group, 0)

        jl = (n_groups - 1) * _NBUF
        for b in range(_NBUF):
            step(jl + b, b, first=False, last=(b + _LEAD >= _NBUF))

        for b in range(_NBUF):
            drain_out(jl + b, b)

    return gather


_gather = _make_gather(4096, 50)


def kernel(x, input_embedding_table):
    b, h = x.shape
    idx = x.reshape(_NW, (b * h) // (_NW * _BPC * h), _BPC * h).astype(jnp.int32)
    return _gather(idx, input_embedding_table)


# speedup vs baseline: 5.9544x; 1.0020x over previous
import functools

import jax
import jax.numpy as jnp
from jax import lax
from jax.experimental import pallas as pl
from jax.experimental.pallas import tpu as pltpu
from jax.experimental.pallas import tpu_sc as plsc

_D = 128
_NW = 32
_NBUF = 4
_LEAD = 2
_BPC = 2


def _make_gather(batch, hist):
    rows_per_w = batch * hist // _NW
    b_per_w = batch // _NW
    chunk = _BPC * hist
    n_chunks = rows_per_w // chunk
    n_groups = n_chunks // _NBUF
    mesh = plsc.VectorSubcoreMesh(core_axis_name="c", subcore_axis_name="s")

    @functools.partial(
        pl.kernel,
        mesh=mesh,
        out_type=jax.ShapeDtypeStruct((batch, hist, _D), jnp.float32),
        scratch_types=[
            pltpu.VMEM((n_chunks, chunk), jnp.int32),
            pltpu.VMEM((_NBUF, chunk, _D), jnp.float32),
        ]
        + [pltpu.SemaphoreType.DMA] * (2 * _NBUF),
        compiler_params=pltpu.CompilerParams(use_tc_tiling_on_sc=True),
    )
    def gather(idx_hbm, tbl_hbm, out_hbm, idx_v, bufs, *sems):
        gsem = sems[:_NBUF]
        osem = sems[_NBUF:]
        wid = lax.axis_index("s") * 2 + lax.axis_index("c")
        pltpu.sync_copy(idx_hbm.at[wid], idx_v)
        b_base = wid * b_per_w

        def fire_gather(j, b):
            pltpu.async_copy(tbl_hbm.at[idx_v.at[j]], bufs.at[b], gsem[b])

        def drain_gather(j, b):
            pltpu.make_async_copy(tbl_hbm.at[idx_v.at[j]], bufs.at[b],
                                  gsem[b]).wait()

        def fire_out(j, b):
            for r in range(_BPC):
                pltpu.async_copy(bufs.at[b, pl.ds(r * hist, hist)],
                                 out_hbm.at[b_base + j * _BPC + r],
                                 osem[b])

        def drain_out(j, b):
            for r in range(_BPC):
                pltpu.make_async_copy(bufs.at[b, pl.ds(r * hist, hist)],
                                      out_hbm.at[b_base + j * _BPC + r],
                                      osem[b]).wait()

        for b in range(_LEAD):
            fire_gather(b, b)

        def step(j, b, first, last):
            c = (b + _LEAD) % _NBUF
            if not last:
                if not first:
                    drain_out(j + _LEAD - _NBUF, c)
                fire_gather(j + _LEAD, c)
            drain_gather(j, b)
            fire_out(j, b)

        for b in range(_NBUF):
            step(b, b, first=(b + _LEAD < _NBUF), last=False)

        def group(i, carry):
            j0 = i * _NBUF
            for b in range(_NBUF):
                step(j0 + b, b, first=False, last=False)
            return carry

        lax.fori_loop(1, n_groups - 1, group, 0)

        jl = (n_groups - 1) * _NBUF
        for b in range(_NBUF):
            step(jl + b, b, first=False, last=(b + _LEAD >= _NBUF))

        for b in range(_NBUF):
            drain_out(jl + b, b)

    return gather


_gather = _make_gather(4096, 50)


def kernel(x, input_embedding_table):
    b, h = x.shape
    idx = x.reshape(_NW, (b * h) // (_NW * _BPC * h), _BPC * h).astype(jnp.int32)
    return _gather(idx, input_embedding_table)


# speedup vs baseline: 10.5183x; 1.7665x over previous
import functools

import jax
import jax.numpy as jnp
from jax import lax
from jax.experimental import pallas as pl
from jax.experimental.pallas import tpu as pltpu
from jax.experimental.pallas import tpu_sc as plsc

_D = 128
_NW = 32
_CHUNK = 128
_NBUF = 5
_LEAD = 2


def _make_gather(b_total):
    rows_per_w = b_total // _NW
    n_chunks = rows_per_w // _CHUNK
    n_groups = n_chunks // _NBUF
    mesh = plsc.VectorSubcoreMesh(core_axis_name="c", subcore_axis_name="s")

    @functools.partial(
        pl.kernel,
        mesh=mesh,
        out_type=jax.ShapeDtypeStruct((b_total, _D), jnp.float32),
        scratch_types=[
            pltpu.VMEM((n_chunks, _CHUNK), jnp.int32),
            pltpu.VMEM((_NBUF, _CHUNK, _D), jnp.float32),
        ]
        + [pltpu.SemaphoreType.DMA] * (2 * _NBUF),
    )
    def gather(idx_hbm, tbl_hbm, out_hbm, idx_v, bufs, *sems):
        gsem = sems[:_NBUF]
        osem = sems[_NBUF:]
        wid = lax.axis_index("s") * 2 + lax.axis_index("c")
        pltpu.sync_copy(idx_hbm.at[wid], idx_v)
        base = wid * rows_per_w

        def fire_gather(j, b):
            pltpu.async_copy(tbl_hbm.at[idx_v.at[j]], bufs.at[b], gsem[b])

        def drain_gather(j, b):
            pltpu.make_async_copy(tbl_hbm.at[idx_v.at[j]], bufs.at[b],
                                  gsem[b]).wait()

        def fire_out(j, b):
            pltpu.async_copy(bufs.at[b],
                             out_hbm.at[pl.ds(base + j * _CHUNK, _CHUNK)],
                             osem[b])

        def drain_out(j, b):
            pltpu.make_async_copy(bufs.at[b],
                                  out_hbm.at[pl.ds(base + j * _CHUNK, _CHUNK)],
                                  osem[b]).wait()

        for b in range(_LEAD):
            fire_gather(b, b)

        def step(j, b, first, last):
            c = (b + _LEAD) % _NBUF
            if not last:
                if not first:
                    drain_out(j + _LEAD - _NBUF, c)
                fire_gather(j + _LEAD, c)
            drain_gather(j, b)
            fire_out(j, b)

        for b in range(_NBUF):
            step(b, b, first=(b + _LEAD < _NBUF), last=False)

        def group(i, carry):
            j0 = i * _NBUF
            for b in range(_NBUF):
                step(j0 + b, b, first=False, last=False)
            return carry

        lax.fori_loop(1, n_groups - 1, group, 0)

        jl = (n_groups - 1) * _NBUF
        for b in range(_NBUF):
            step(jl + b, b, first=False, last=(b + _LEAD >= _NBUF))

        for b in range(_NBUF):
            drain_out(jl + b, b)

    return gather


_gather = _make_gather(4096 * 50)


def kernel(x, input_embedding_table):
    b, h = x.shape
    idx = x.T.reshape(_NW, (b * h) // (_NW * _CHUNK), _CHUNK).astype(jnp.int32)
    out = _gather(idx, input_embedding_table)
    return out.reshape(h, b, _D).transpose(1, 0, 2)
